# Initial kernel scaffold; baseline (speedup 1.0000x reference)
import jax
import jax.numpy as jnp
from jax.experimental import pallas as pl


def kernel(inputs):
    raise NotImplementedError("write your pallas kernel here")


# speedup vs baseline: 550.6508x; 550.6508x over previous
import jax
import jax.numpy as jnp
from jax.experimental import pallas as pl


def _body(x_ref, o_ref):
    x = x_ref[...]
    in_table = (x >= 0) & (x < 1000)
    y = jnp.where(in_table, x + 2, jnp.ones_like(x))
    o_ref[...] = jnp.where(x == 1, jnp.zeros_like(x), y)


def kernel(inputs):
    inputs = inputs.astype(jnp.int32)
    n, m = inputs.shape
    block_rows = 1024
    grid = n // block_rows
    return pl.pallas_call(
        _body,
        grid=(grid,),
        in_specs=[pl.BlockSpec((block_rows, m), lambda i: (i, 0))],
        out_specs=pl.BlockSpec((block_rows, m), lambda i: (i, 0)),
        out_shape=jax.ShapeDtypeStruct((n, m), jnp.int32),
    )(inputs)
